# baseline (speedup 1.0000x reference)
import functools

import jax
import jax.numpy as jnp
from jax import lax
from jax.experimental import pallas as pl
from jax.experimental.pallas import tpu as pltpu
from jax.experimental.pallas import tpu_sc as plsc

_N = 10000
_E = 320000
_D = 128

_NC = 2
_NS = 16
_NW = _NC * _NS
_EPT = _E // _NW
_CH = 100
_NCHUNK = _EPT // _CH
_SB = 20
_NSC = _NCHUNK // _SB
_NP = 10240
_RPS = _NP // _NS
_ZR = 16
_ZRD = 128
_DW = 16

_mesh = plsc.VectorSubcoreMesh(core_axis_name="c", subcore_axis_name="s")


@functools.partial(
    pl.kernel,
    out_type=jax.ShapeDtypeStruct((_NC, _NP, _DW), jnp.float32),
    mesh=_mesh,
    scratch_types=[
        pltpu.VMEM((_SB, _CH), jnp.int32),
        pltpu.VMEM((_CH, _DW), jnp.float32),
        pltpu.VMEM((_ZRD, _DW), jnp.float32),
        pltpu.VMEM_SHARED((_NP, _DW), jnp.float32),
    ],
)
def _sc_degree(dst_hbm, out_hbm, didx, ones_v, zb, acc):
    c = lax.axis_index("c")
    s = lax.axis_index("s")
    wid = s * _NC + c

    one = jnp.full((16,), 1.0, dtype=jnp.float32)
    zero = jnp.zeros((16,), dtype=jnp.float32)
    for r in range(_CH):
        ones_v[r, :] = one
    for r in range(_ZRD):
        zb[r, :] = zero

    for j in range(_RPS // _ZRD):
        pltpu.sync_copy(zb, acc.at[pl.ds(s * _RPS + j * _ZRD, _ZRD)])
    plsc.subcore_barrier()

    for sc in range(_NSC):
        pltpu.sync_copy(dst_hbm.at[wid * _NSC + sc], didx)

        @pl.loop(0, _SB)
        def _(ci):
            pltpu.sync_copy(ones_v, acc.at[didx.at[ci]], add=True)

    plsc.subcore_barrier()
    pltpu.sync_copy(acc.at[pl.ds(s * _RPS, _RPS)],
                    out_hbm.at[c, pl.ds(s * _RPS, _RPS)])


@functools.partial(
    pl.kernel,
    out_type=jax.ShapeDtypeStruct((_NC, _NP, _D), jnp.float32),
    mesh=_mesh,
    scratch_types=[
        pltpu.VMEM((_SB, _CH), jnp.int32),
        pltpu.VMEM((_SB, _CH), jnp.int32),
        pltpu.VMEM((_SB, _CH), jnp.int32),
        pltpu.VMEM((_SB, _CH), jnp.int32),
        pltpu.VMEM((_CH, _D), jnp.float32),
        pltpu.VMEM((_CH, _D), jnp.float32),
        pltpu.VMEM((_ZR, _D), jnp.float32),
        pltpu.VMEM_SHARED((_NP, _D), jnp.float32),
        pltpu.SemaphoreType.DMA,
        pltpu.SemaphoreType.DMA,
        pltpu.SemaphoreType.DMA,
    ],
)
def _sc_scatter(g_hbm, src_hbm, dst_hbm, out_hbm,
                sidxA, sidxB, didxA, didxB, rows0, rows1, zb, acc,
                sem0, sem1, semi):
    c = lax.axis_index("c")
    s = lax.axis_index("s")
    wid = s * _NC + c

    zero = jnp.zeros((16,), dtype=jnp.float32)
    for r in range(_ZR):
        for k in range(_D // 16):
            zb[r, pl.ds(k * 16, 16)] = zero

    for j in range(_RPS // _ZR):
        pltpu.sync_copy(zb, acc.at[pl.ds(s * _RPS + j * _ZR, _ZR)])
    plsc.subcore_barrier()

    pltpu.sync_copy(src_hbm.at[wid * _NSC], sidxA)
    pltpu.sync_copy(dst_hbm.at[wid * _NSC], didxA)
    pltpu.async_copy(g_hbm.at[sidxA.at[0]], rows0, sem0)

    for sc in range(_NSC):
        sidx, didx = (sidxA, didxA) if sc % 2 == 0 else (sidxB, didxB)
        nsidx, ndidx = (sidxB, didxB) if sc % 2 == 0 else (sidxA, didxA)
        if sc + 1 < _NSC:
            pltpu.async_copy(src_hbm.at[wid * _NSC + sc + 1], nsidx, semi)
            pltpu.async_copy(dst_hbm.at[wid * _NSC + sc + 1], ndidx, semi)

        @pl.loop(0, _SB // 2 - 1)
        def _(pi):
            c0 = pi * 2
            pltpu.async_copy(g_hbm.at[sidx.at[c0 + 1]], rows1, sem1)
            pltpu.make_async_copy(g_hbm.at[sidx.at[c0]], rows0, sem0).wait()
            pltpu.sync_copy(rows0, acc.at[didx.at[c0]], add=True)
            pltpu.async_copy(g_hbm.at[sidx.at[c0 + 2]], rows0, sem0)
            pltpu.make_async_copy(g_hbm.at[sidx.at[c0 + 1]], rows1, sem1).wait()
            pltpu.sync_copy(rows1, acc.at[didx.at[c0 + 1]], add=True)

        pltpu.async_copy(g_hbm.at[sidx.at[_SB - 1]], rows1, sem1)
        pltpu.make_async_copy(g_hbm.at[sidx.at[_SB - 2]], rows0, sem0).wait()
        pltpu.sync_copy(rows0, acc.at[didx.at[_SB - 2]], add=True)
        if sc + 1 < _NSC:
            pltpu.make_async_copy(
                src_hbm.at[wid * _NSC + sc + 1], nsidx, semi).wait()
            pltpu.make_async_copy(
                dst_hbm.at[wid * _NSC + sc + 1], ndidx, semi).wait()
            pltpu.async_copy(g_hbm.at[nsidx.at[0]], rows0, sem0)
        pltpu.make_async_copy(g_hbm.at[sidx.at[_SB - 1]], rows1, sem1).wait()
        pltpu.sync_copy(rows1, acc.at[didx.at[_SB - 1]], add=True)

    plsc.subcore_barrier()
    pltpu.sync_copy(acc.at[pl.ds(s * _RPS, _RPS)],
                    out_hbm.at[c, pl.ds(s * _RPS, _RPS)])


_R = 1000
_G = _N // _R


def _dinv_block(dp_ref):
    deg = dp_ref[0, :, 0:1] + dp_ref[1, :, 0:1] + 1.0
    return lax.rsqrt(deg)


def _tc_pre(x_ref, w0_ref, w1_ref, dp_ref, a0_ref, g_ref):
    a0 = a0_ref[0, 0]
    h = jnp.dot(x_ref[...], w0_ref[...], preferred_element_type=jnp.float32)
    h = jnp.where(h >= 0, h, a0 * h)
    g = jnp.dot(h, w1_ref[...], preferred_element_type=jnp.float32)
    g_ref[...] = g * _dinv_block(dp_ref)


def _tc_mid(tp_ref, g_ref, dp_ref, w_ref, a_ref, o_ref):
    a = a_ref[0, 0]
    dinv = _dinv_block(dp_ref)
    h = (tp_ref[0] + tp_ref[1] + g_ref[...]) * dinv
    h = jnp.where(h >= 0, h, a * h)
    o = jnp.dot(h, w_ref[...], preferred_element_type=jnp.float32)
    o_ref[...] = o * dinv


def _tc_post(tp_ref, g_ref, dp_ref, w_ref, a_ref, b_ref, o_ref):
    a = a_ref[0, 0]
    dinv = _dinv_block(dp_ref)
    h = (tp_ref[0] + tp_ref[1] + g_ref[...]) * dinv
    h = jnp.where(h >= 0, h, a * h)
    o_ref[...] = jnp.dot(h, w_ref[...],
                         preferred_element_type=jnp.float32) + b_ref[...]


_b_rows = pl.BlockSpec((_R, _D), lambda i: (i, 0))
_b_w = pl.BlockSpec((_D, _D), lambda i: (0, 0))
_b_dp = pl.BlockSpec((_NC, _R, _DW), lambda i: (0, i, 0))
_b_tp = pl.BlockSpec((_NC, _R, _D), lambda i: (0, i, 0))
_b_scal = pl.BlockSpec((1, 1), lambda i: (0, 0))
_b_bias = pl.BlockSpec((1, _D), lambda i: (0, 0))
_out_rows = jax.ShapeDtypeStruct((_N, _D), jnp.float32)

_pre_call = pl.pallas_call(
    _tc_pre, grid=(_G,),
    in_specs=[_b_rows, _b_w, _b_w, _b_dp, _b_scal],
    out_specs=_b_rows, out_shape=_out_rows)

_mid_call = pl.pallas_call(
    _tc_mid, grid=(_G,),
    in_specs=[_b_tp, _b_rows, _b_dp, _b_w, _b_scal],
    out_specs=_b_rows, out_shape=_out_rows)

_post_call = pl.pallas_call(
    _tc_post, grid=(_G,),
    in_specs=[_b_tp, _b_rows, _b_dp, _b_w, _b_scal, _b_bias],
    out_specs=_b_rows, out_shape=_out_rows)


def kernel(x, edge_index, W0, a0, W1, a1, W2, a2, W3, b3):
    src = edge_index[0].astype(jnp.int32).reshape(_NW * _NSC, _SB, _CH)
    dst = edge_index[1].astype(jnp.int32).reshape(_NW * _NSC, _SB, _CH)

    degp = _sc_degree(dst)
    a0r = a0.reshape(1, 1)
    a1r = a1.reshape(1, 1)
    a2r = a2.reshape(1, 1)
    b3r = b3.reshape(1, _D)

    g1 = _pre_call(x, W0, W1, degp, a0r)
    t1 = _sc_scatter(g1, src, dst)
    g2 = _mid_call(t1, g1, degp, W2, a1r)
    t2 = _sc_scatter(g2, src, dst)
    out = _post_call(t2, g2, degp, W3, a2r, b3r)
    return out


# speedup vs baseline: 1.0751x; 1.0751x over previous
import functools

import jax
import jax.numpy as jnp
from jax import lax
from jax.experimental import pallas as pl
from jax.experimental.pallas import tpu as pltpu
from jax.experimental.pallas import tpu_sc as plsc

_N = 10000
_E = 320000
_D = 128

_NC = 2
_NS = 16
_NW = _NC * _NS
_EPT = _E // _NW
_CH = 80
_NCHUNK = _EPT // _CH
_SB = 25
_NSC = _NCHUNK // _SB
_NP = 10112
_RPS = _NP // _NS
_ZR = 16
_ZRD = 128
_DW = 16

_mesh = plsc.VectorSubcoreMesh(core_axis_name="c", subcore_axis_name="s")


@functools.partial(
    pl.kernel,
    out_type=jax.ShapeDtypeStruct((_NC, _NP, _DW), jnp.float32),
    mesh=_mesh,
    scratch_types=[
        pltpu.VMEM((_SB, _CH), jnp.int32),
        pltpu.VMEM((_CH, _DW), jnp.float32),
        pltpu.VMEM((_ZRD, _DW), jnp.float32),
        pltpu.VMEM_SHARED((_NP, _DW), jnp.float32),
    ],
)
def _sc_degree(dst_hbm, out_hbm, didx, ones_v, zb, acc):
    c = lax.axis_index("c")
    s = lax.axis_index("s")
    wid = s * _NC + c

    one = jnp.full((16,), 1.0, dtype=jnp.float32)
    zero = jnp.zeros((16,), dtype=jnp.float32)
    for r in range(_CH):
        ones_v[r, :] = one
    for r in range(_ZRD):
        zb[r, :] = zero

    for j in range(_RPS // _ZRD):
        pltpu.sync_copy(zb, acc.at[pl.ds(s * _RPS + j * _ZRD, _ZRD)])
    if _RPS % _ZRD:
        pltpu.sync_copy(
            zb.at[pl.ds(0, _RPS % _ZRD)],
            acc.at[pl.ds(s * _RPS + (_RPS // _ZRD) * _ZRD, _RPS % _ZRD)])
    plsc.subcore_barrier()

    for sc in range(_NSC):
        pltpu.sync_copy(dst_hbm.at[wid * _NSC + sc], didx)

        @pl.loop(0, _SB)
        def _(ci):
            pltpu.sync_copy(ones_v, acc.at[didx.at[ci]], add=True)

    plsc.subcore_barrier()
    pltpu.sync_copy(acc.at[pl.ds(s * _RPS, _RPS)],
                    out_hbm.at[c, pl.ds(s * _RPS, _RPS)])


@functools.partial(
    pl.kernel,
    out_type=jax.ShapeDtypeStruct((_NC, _NP, _D), jnp.float32),
    mesh=_mesh,
    scratch_types=[
        pltpu.VMEM((_SB, _CH), jnp.int32),
        pltpu.VMEM((_SB, _CH), jnp.int32),
        pltpu.VMEM((_SB, _CH), jnp.int32),
        pltpu.VMEM((_SB, _CH), jnp.int32),
        pltpu.VMEM((_CH, _D), jnp.float32),
        pltpu.VMEM((_CH, _D), jnp.float32),
        pltpu.VMEM((_CH, _D), jnp.float32),
        pltpu.VMEM((_ZR, _D), jnp.float32),
        pltpu.VMEM_SHARED((_NP, _D), jnp.float32),
        pltpu.SemaphoreType.DMA,
        pltpu.SemaphoreType.DMA,
        pltpu.SemaphoreType.DMA,
        pltpu.SemaphoreType.DMA,
        pltpu.SemaphoreType.DMA,
        pltpu.SemaphoreType.DMA,
        pltpu.SemaphoreType.DMA,
    ],
)
def _sc_scatter(g_hbm, src_hbm, dst_hbm, out_hbm,
                sidxA, sidxB, didxA, didxB, rows0, rows1, rows2, zb, acc,
                gs0, gs1, gs2, ss0, ss1, ss2, semi):
    c = lax.axis_index("c")
    s = lax.axis_index("s")
    wid = s * _NC + c

    zero = jnp.zeros((16,), dtype=jnp.float32)
    for r in range(_ZR):
        for k in range(_D // 16):
            zb[r, pl.ds(k * 16, 16)] = zero

    for j in range(_RPS // _ZR):
        pltpu.sync_copy(zb, acc.at[pl.ds(s * _RPS + j * _ZR, _ZR)])
    if _RPS % _ZR:
        pltpu.sync_copy(
            zb.at[pl.ds(0, _RPS % _ZR)],
            acc.at[pl.ds(s * _RPS + (_RPS // _ZR) * _ZR, _RPS % _ZR)])
    plsc.subcore_barrier()

    rowsb = (rows0, rows1, rows2)
    gsem = (gs0, gs1, gs2)
    ssem = (ss0, ss1, ss2)
    sslab = (sidxA, sidxB)
    dslab = (didxA, didxB)

    def src_hslab(k):
        return src_hbm.at[wid * _NSC + k]

    def dst_hslab(k):
        return dst_hbm.at[wid * _NSC + k]

    def gath(ci):
        k, j, b = ci // _SB, ci % _SB, ci % 3
        return g_hbm.at[sslab[k % 2].at[j]], rowsb[b], gsem[b]

    def scat(ci):
        k, j, b = ci // _SB, ci % _SB, ci % 3
        return rowsb[b], acc.at[dslab[k % 2].at[j]], ssem[b]

    pltpu.sync_copy(src_hslab(0), sidxA)
    pltpu.sync_copy(dst_hslab(0), didxA)
    pltpu.async_copy(*gath(0))
    pltpu.async_copy(*gath(1))

    for ci in range(_NCHUNK):
        k = ci // _SB
        if ci % _SB == 1 and k + 1 < _NSC:
            pltpu.async_copy(src_hslab(k + 1), sslab[(k + 1) % 2], semi)
            pltpu.async_copy(dst_hslab(k + 1), dslab[(k + 1) % 2], semi)
        ga, gb, gc = gath(ci)
        pltpu.make_async_copy(ga, gb, gc).wait()
        sa, sb, sc_ = scat(ci)
        pltpu.async_copy(sa, sb, sc_, add=True)
        if 1 <= ci <= _NCHUNK - 2:
            pa, pb, pc = scat(ci - 1)
            pltpu.make_async_copy(pa, pb, pc).wait()
        if ci + 2 < _NCHUNK:
            if (ci + 2) % _SB == 0:
                kk = (ci + 2) // _SB
                pltpu.make_async_copy(
                    src_hslab(kk), sslab[kk % 2], semi).wait()
                pltpu.make_async_copy(
                    dst_hslab(kk), dslab[kk % 2], semi).wait()
            pltpu.async_copy(*gath(ci + 2))

    for ci in (_NCHUNK - 2, _NCHUNK - 1):
        da, db, dc = scat(ci)
        pltpu.make_async_copy(da, db, dc).wait()

    plsc.subcore_barrier()
    pltpu.sync_copy(acc.at[pl.ds(s * _RPS, _RPS)],
                    out_hbm.at[c, pl.ds(s * _RPS, _RPS)])


_R = 1000
_G = _N // _R


def _dinv_block(dp_ref):
    deg = dp_ref[0, :, 0:1] + dp_ref[1, :, 0:1] + 1.0
    return lax.rsqrt(deg)


def _tc_pre(x_ref, w0_ref, w1_ref, dp_ref, a0_ref, g_ref):
    a0 = a0_ref[0, 0]
    h = jnp.dot(x_ref[...], w0_ref[...], preferred_element_type=jnp.float32)
    h = jnp.where(h >= 0, h, a0 * h)
    g = jnp.dot(h, w1_ref[...], preferred_element_type=jnp.float32)
    g_ref[...] = g * _dinv_block(dp_ref)


def _tc_mid(tp_ref, g_ref, dp_ref, w_ref, a_ref, o_ref):
    a = a_ref[0, 0]
    dinv = _dinv_block(dp_ref)
    h = (tp_ref[0] + tp_ref[1] + g_ref[...]) * dinv
    h = jnp.where(h >= 0, h, a * h)
    o = jnp.dot(h, w_ref[...], preferred_element_type=jnp.float32)
    o_ref[...] = o * dinv


def _tc_post(tp_ref, g_ref, dp_ref, w_ref, a_ref, b_ref, o_ref):
    a = a_ref[0, 0]
    dinv = _dinv_block(dp_ref)
    h = (tp_ref[0] + tp_ref[1] + g_ref[...]) * dinv
    h = jnp.where(h >= 0, h, a * h)
    o_ref[...] = jnp.dot(h, w_ref[...],
                         preferred_element_type=jnp.float32) + b_ref[...]


_b_rows = pl.BlockSpec((_R, _D), lambda i: (i, 0))
_b_w = pl.BlockSpec((_D, _D), lambda i: (0, 0))
_b_dp = pl.BlockSpec((_NC, _R, _DW), lambda i: (0, i, 0))
_b_tp = pl.BlockSpec((_NC, _R, _D), lambda i: (0, i, 0))
_b_scal = pl.BlockSpec((1, 1), lambda i: (0, 0))
_b_bias = pl.BlockSpec((1, _D), lambda i: (0, 0))
_out_rows = jax.ShapeDtypeStruct((_N, _D), jnp.float32)

_pre_call = pl.pallas_call(
    _tc_pre, grid=(_G,),
    in_specs=[_b_rows, _b_w, _b_w, _b_dp, _b_scal],
    out_specs=_b_rows, out_shape=_out_rows)

_mid_call = pl.pallas_call(
    _tc_mid, grid=(_G,),
    in_specs=[_b_tp, _b_rows, _b_dp, _b_w, _b_scal],
    out_specs=_b_rows, out_shape=_out_rows)

_post_call = pl.pallas_call(
    _tc_post, grid=(_G,),
    in_specs=[_b_tp, _b_rows, _b_dp, _b_w, _b_scal, _b_bias],
    out_specs=_b_rows, out_shape=_out_rows)


def kernel(x, edge_index, W0, a0, W1, a1, W2, a2, W3, b3):
    src = edge_index[0].astype(jnp.int32).reshape(_NW * _NSC, _SB, _CH)
    dst = edge_index[1].astype(jnp.int32).reshape(_NW * _NSC, _SB, _CH)

    degp = _sc_degree(dst)
    a0r = a0.reshape(1, 1)
    a1r = a1.reshape(1, 1)
    a2r = a2.reshape(1, 1)
    b3r = b3.reshape(1, _D)

    g1 = _pre_call(x, W0, W1, degp, a0r)
    t1 = _sc_scatter(g1, src, dst)
    g2 = _mid_call(t1, g1, degp, W2, a1r)
    t2 = _sc_scatter(g2, src, dst)
    out = _post_call(t2, g2, degp, W3, a2r, b3r)
    return out


# speedup vs baseline: 1.1032x; 1.0261x over previous
import functools

import jax
import jax.numpy as jnp
from jax import lax
from jax.experimental import pallas as pl
from jax.experimental.pallas import tpu as pltpu
from jax.experimental.pallas import tpu_sc as plsc

_N = 10000
_E = 320000
_D = 128

_NC = 2
_NS = 16
_NW = _NC * _NS
_EPT = _E // _NW
_CH = 80
_NCHUNK = _EPT // _CH
_SB = 25
_NSC = _NCHUNK // _SB
_NP = 10112
_RPS = _NP // _NS
_ZR = 16
_ZRD = 128
_DW = 16

_mesh = plsc.VectorSubcoreMesh(core_axis_name="c", subcore_axis_name="s")


@functools.partial(
    pl.kernel,
    out_type=jax.ShapeDtypeStruct((_NC, _NP, _DW), jnp.float32),
    mesh=_mesh,
    scratch_types=[
        pltpu.VMEM((_SB, _CH), jnp.int32),
        pltpu.VMEM((_SB, _CH), jnp.int32),
        pltpu.VMEM((_CH, _DW), jnp.float32),
        pltpu.VMEM((_ZRD, _DW), jnp.float32),
        pltpu.VMEM_SHARED((_NP, _DW), jnp.float32),
        pltpu.SemaphoreType.DMA,
        pltpu.SemaphoreType.DMA,
    ],
)
def _sc_degree(dst_hbm, out_hbm, didxA, didxB, ones_v, zb, acc, sems, semi):
    c = lax.axis_index("c")
    s = lax.axis_index("s")
    wid = s * _NC + c

    one = jnp.full((16,), 1.0, dtype=jnp.float32)
    zero = jnp.zeros((16,), dtype=jnp.float32)
    for r in range(_CH):
        ones_v[r, :] = one
    for r in range(_ZRD):
        zb[r, :] = zero

    for j in range(_RPS // _ZRD):
        pltpu.sync_copy(zb, acc.at[pl.ds(s * _RPS + j * _ZRD, _ZRD)])
    if _RPS % _ZRD:
        pltpu.sync_copy(
            zb.at[pl.ds(0, _RPS % _ZRD)],
            acc.at[pl.ds(s * _RPS + (_RPS // _ZRD) * _ZRD, _RPS % _ZRD)])
    plsc.subcore_barrier()

    dslab = (didxA, didxB)
    pltpu.sync_copy(dst_hbm.at[wid * _NSC], didxA)
    for k in range(_NSC):
        if k + 1 < _NSC:
            pltpu.async_copy(dst_hbm.at[wid * _NSC + k + 1],
                             dslab[(k + 1) % 2], semi)
        for j in range(_SB):
            pltpu.async_copy(ones_v, acc.at[dslab[k % 2].at[j]], sems,
                             add=True)
        for j in range(_SB):
            pltpu.make_async_copy(ones_v, acc.at[dslab[k % 2].at[j]],
                                  sems).wait()
        if k + 1 < _NSC:
            pltpu.make_async_copy(dst_hbm.at[wid * _NSC + k + 1],
                                  dslab[(k + 1) % 2], semi).wait()

    plsc.subcore_barrier()
    pltpu.sync_copy(acc.at[pl.ds(s * _RPS, _RPS)],
                    out_hbm.at[c, pl.ds(s * _RPS, _RPS)])


@functools.partial(
    pl.kernel,
    out_type=jax.ShapeDtypeStruct((_NC, _NP, _D), jnp.float32),
    mesh=_mesh,
    scratch_types=[
        pltpu.VMEM((_SB, _CH), jnp.int32),
        pltpu.VMEM((_SB, _CH), jnp.int32),
        pltpu.VMEM((_SB, _CH), jnp.int32),
        pltpu.VMEM((_SB, _CH), jnp.int32),
        pltpu.VMEM((_CH, _D), jnp.float32),
        pltpu.VMEM((_CH, _D), jnp.float32),
        pltpu.VMEM((_CH, _D), jnp.float32),
        pltpu.VMEM((_ZR, _D), jnp.float32),
        pltpu.VMEM_SHARED((_NP, _D), jnp.float32),
        pltpu.SemaphoreType.DMA,
        pltpu.SemaphoreType.DMA,
        pltpu.SemaphoreType.DMA,
        pltpu.SemaphoreType.DMA,
        pltpu.SemaphoreType.DMA,
        pltpu.SemaphoreType.DMA,
        pltpu.SemaphoreType.DMA,
    ],
)
def _sc_scatter(g_hbm, src_hbm, dst_hbm, out_hbm,
                sidxA, sidxB, didxA, didxB, rows0, rows1, rows2, zb, acc,
                gs0, gs1, gs2, ss0, ss1, ss2, semi):
    c = lax.axis_index("c")
    s = lax.axis_index("s")
    wid = s * _NC + c

    zero = jnp.zeros((16,), dtype=jnp.float32)
    for r in range(_ZR):
        for k in range(_D // 16):
            zb[r, pl.ds(k * 16, 16)] = zero

    for j in range(_RPS // _ZR):
        pltpu.sync_copy(zb, acc.at[pl.ds(s * _RPS + j * _ZR, _ZR)])
    if _RPS % _ZR:
        pltpu.sync_copy(
            zb.at[pl.ds(0, _RPS % _ZR)],
            acc.at[pl.ds(s * _RPS + (_RPS // _ZR) * _ZR, _RPS % _ZR)])
    plsc.subcore_barrier()

    rowsb = (rows0, rows1, rows2)
    gsem = (gs0, gs1, gs2)
    ssem = (ss0, ss1, ss2)
    sslab = (sidxA, sidxB)
    dslab = (didxA, didxB)

    def src_hslab(k):
        return src_hbm.at[wid * _NSC + k]

    def dst_hslab(k):
        return dst_hbm.at[wid * _NSC + k]

    def gath(ci):
        k, j, b = ci // _SB, ci % _SB, ci % 3
        return g_hbm.at[sslab[k % 2].at[j]], rowsb[b], gsem[b]

    def scat(ci):
        k, j, b = ci // _SB, ci % _SB, ci % 3
        return rowsb[b], acc.at[dslab[k % 2].at[j]], ssem[b]

    pltpu.sync_copy(src_hslab(0), sidxA)
    pltpu.sync_copy(dst_hslab(0), didxA)
    pltpu.async_copy(*gath(0))
    pltpu.async_copy(*gath(1))

    for ci in range(_NCHUNK):
        k = ci // _SB
        if ci % _SB == 1 and k + 1 < _NSC:
            pltpu.async_copy(src_hslab(k + 1), sslab[(k + 1) % 2], semi)
            pltpu.async_copy(dst_hslab(k + 1), dslab[(k + 1) % 2], semi)
        ga, gb, gc = gath(ci)
        pltpu.make_async_copy(ga, gb, gc).wait()
        sa, sb, sc_ = scat(ci)
        pltpu.async_copy(sa, sb, sc_, add=True)
        if 1 <= ci <= _NCHUNK - 2:
            pa, pb, pc = scat(ci - 1)
            pltpu.make_async_copy(pa, pb, pc).wait()
        if ci + 2 < _NCHUNK:
            if (ci + 2) % _SB == 0:
                kk = (ci + 2) // _SB
                pltpu.make_async_copy(
                    src_hslab(kk), sslab[kk % 2], semi).wait()
                pltpu.make_async_copy(
                    dst_hslab(kk), dslab[kk % 2], semi).wait()
            pltpu.async_copy(*gath(ci + 2))

    for ci in (_NCHUNK - 2, _NCHUNK - 1):
        da, db, dc = scat(ci)
        pltpu.make_async_copy(da, db, dc).wait()

    plsc.subcore_barrier()
    pltpu.sync_copy(acc.at[pl.ds(s * _RPS, _RPS)],
                    out_hbm.at[c, pl.ds(s * _RPS, _RPS)])


_R = 1000
_G = _N // _R


def _dinv_block(dp_ref):
    deg = dp_ref[0, :, 0:1] + dp_ref[1, :, 0:1] + 1.0
    return lax.rsqrt(deg)


def _tc_pre(x_ref, w0_ref, w1_ref, dp_ref, a0_ref, g_ref):
    a0 = a0_ref[0, 0]
    h = jnp.dot(x_ref[...], w0_ref[...], preferred_element_type=jnp.float32)
    h = jnp.where(h >= 0, h, a0 * h)
    g = jnp.dot(h, w1_ref[...], preferred_element_type=jnp.float32)
    g_ref[...] = g * _dinv_block(dp_ref)


def _tc_mid(tp_ref, g_ref, dp_ref, w_ref, a_ref, o_ref):
    a = a_ref[0, 0]
    dinv = _dinv_block(dp_ref)
    h = (tp_ref[0] + tp_ref[1] + g_ref[...]) * dinv
    h = jnp.where(h >= 0, h, a * h)
    o = jnp.dot(h, w_ref[...], preferred_element_type=jnp.float32)
    o_ref[...] = o * dinv


def _tc_post(tp_ref, g_ref, dp_ref, w_ref, a_ref, b_ref, o_ref):
    a = a_ref[0, 0]
    dinv = _dinv_block(dp_ref)
    h = (tp_ref[0] + tp_ref[1] + g_ref[...]) * dinv
    h = jnp.where(h >= 0, h, a * h)
    o_ref[...] = jnp.dot(h, w_ref[...],
                         preferred_element_type=jnp.float32) + b_ref[...]


_b_rows = pl.BlockSpec((_R, _D), lambda i: (i, 0))
_b_w = pl.BlockSpec((_D, _D), lambda i: (0, 0))
_b_dp = pl.BlockSpec((_NC, _R, _DW), lambda i: (0, i, 0))
_b_tp = pl.BlockSpec((_NC, _R, _D), lambda i: (0, i, 0))
_b_scal = pl.BlockSpec((1, 1), lambda i: (0, 0))
_b_bias = pl.BlockSpec((1, _D), lambda i: (0, 0))
_out_rows = jax.ShapeDtypeStruct((_N, _D), jnp.float32)

_pre_call = pl.pallas_call(
    _tc_pre, grid=(_G,),
    in_specs=[_b_rows, _b_w, _b_w, _b_dp, _b_scal],
    out_specs=_b_rows, out_shape=_out_rows)

_mid_call = pl.pallas_call(
    _tc_mid, grid=(_G,),
    in_specs=[_b_tp, _b_rows, _b_dp, _b_w, _b_scal],
    out_specs=_b_rows, out_shape=_out_rows)

_post_call = pl.pallas_call(
    _tc_post, grid=(_G,),
    in_specs=[_b_tp, _b_rows, _b_dp, _b_w, _b_scal, _b_bias],
    out_specs=_b_rows, out_shape=_out_rows)


def kernel(x, edge_index, W0, a0, W1, a1, W2, a2, W3, b3):
    src = edge_index[0].astype(jnp.int32).reshape(_NW * _NSC, _SB, _CH)
    dst = edge_index[1].astype(jnp.int32).reshape(_NW * _NSC, _SB, _CH)

    degp = _sc_degree(dst)
    a0r = a0.reshape(1, 1)
    a1r = a1.reshape(1, 1)
    a2r = a2.reshape(1, 1)
    b3r = b3.reshape(1, _D)

    g1 = _pre_call(x, W0, W1, degp, a0r)
    t1 = _sc_scatter(g1, src, dst)
    g2 = _mid_call(t1, g1, degp, W2, a1r)
    t2 = _sc_scatter(g2, src, dst)
    out = _post_call(t2, g2, degp, W3, a2r, b3r)
    return out
